# baseline (speedup 1.0000x reference)
import dataclasses
import functools

import jax
import jax.numpy as jnp
from jax import lax
from jax.experimental import pallas as pl
from jax.experimental.pallas import tpu as pltpu
from jax.experimental.pallas import tpu_sc as plsc

_W = 256
_H = 128
_L = 16


def _tanh_body(a_ref, o_ref):
    o_ref[...] = jnp.tanh(a_ref[...])


def _tc_tanh(flat2):
    return pl.pallas_call(
        _tanh_body,
        out_shape=jax.ShapeDtypeStruct(flat2.shape, jnp.float32),
    )(flat2)


def _sc_embed(q_w, fx, fy, embs, simp_w, n, f):
    nwin = n // _W
    mesh = plsc.VectorSubcoreMesh(
        core_axis_name="core", subcore_axis_name="subcore",
        num_cores=2, num_subcores=16,
    )
    cp = pltpu.CompilerParams(use_tc_tiling_on_sc=False)
    if "needs_layout_passes" in pltpu.CompilerParams.__dataclass_fields__:
        cp = dataclasses.replace(cp, needs_layout_passes=False)

    @functools.partial(
        pl.kernel,
        out_type=jax.ShapeDtypeStruct((n // 2, 128), jnp.float32),
        mesh=mesh,
        compiler_params=cp,
        scratch_types=[
            pltpu.VMEM((3 * _W,), jnp.float32),
            pltpu.VMEM((3 * _W,), jnp.float32),
            pltpu.VMEM((3 * _W, 64), jnp.float32),
            pltpu.SemaphoreType.DMA((2,)),
        ],
    )
    def sc_kernel(q_hbm, fx_hbm, fy_hbm, embs_hbm, simp_hbm, out_hbm,
                  cx_v, cy_v, rows_v, sems):
        def body(simp_v, q_v, out_v):
            copies = []
            for h in range(2):
                sem = sems.at[h]
                for c in range(3 * h, 3 * h + 3):
                    idx = simp_v.at[c]
                    d = pl.ds(_H * c, _H)
                    copies.append(pltpu.async_copy(embs_hbm.at[idx], rows_v.at[d], sem))
                    copies.append(pltpu.async_copy(fx_hbm.at[idx], cx_v.at[d], sem))
                    copies.append(pltpu.async_copy(fy_hbm.at[idx], cy_v.at[d], sem))

            for h in range(2):
                for cp_ in copies[9 * h:9 * h + 9]:
                    cp_.wait()

                @pl.loop(_H * h, _H * h + _H, step=_L)
                def _group(b):
                    iot = b + lax.iota(jnp.int32, _L)
                    vrow = 3 * iot

                    v1x = plsc.load_gather(cx_v, [vrow])
                    v2x = plsc.load_gather(cx_v, [vrow + 1])
                    v3x = plsc.load_gather(cx_v, [vrow + 2])
                    v1y = plsc.load_gather(cy_v, [vrow])
                    v2y = plsc.load_gather(cy_v, [vrow + 1])
                    v3y = plsc.load_gather(cy_v, [vrow + 2])
                    p = 2 * iot
                    prow = lax.shift_right_logical(p, 7)
                    pcol = lax.bitwise_and(p, 127)
                    x = plsc.load_gather(q_v, [prow, pcol])
                    y = plsc.load_gather(q_v, [prow, pcol + 1])

                    denom = (v2y - v3y) * (v1x - v3x) + (v3x - v2x) * (v1y - v3y)
                    w1v = ((v2y - v3y) * (x - v3x) + (v3x - v2x) * (y - v3y)) / denom
                    w2v = ((v3y - v1y) * (x - v3x) + (v1x - v3x) * (y - v3y)) / denom
                    w3v = 1.0 - w1v - w2v

                    orow = lax.shift_right_logical(b, 1)
                    for qi in range(_L):
                        w1 = jnp.full((_L,), w1v[qi])
                        w2 = jnp.full((_L,), w2v[qi])
                        w3 = jnp.full((_L,), w3v[qi])
                        r = 3 * b + 3 * qi
                        for fb in range(0, 64, _L):
                            s = pl.ds((qi % 2) * 64 + fb, _L)
                            out_v[orow + qi // 2, s] = (
                                w1 * rows_v[r, pl.ds(fb, _L)]
                                + w2 * rows_v[r + 1, pl.ds(fb, _L)]
                                + w3 * rows_v[r + 2, pl.ds(fb, _L)])

        pltpu.emit_pipeline(
            body,
            grid=(nwin,),
            in_specs=[
                pl.BlockSpec((3 * _W // 128, 128), lambda i: (i, 0)),
                pl.BlockSpec((2 * _W // 128, 128), lambda i: (i, 0)),
            ],
            out_specs=[pl.BlockSpec((_W // 2, 128), lambda i: (i, 0))],
            core_axis_name=("core", "subcore"),
            dimension_semantics=(pltpu.PARALLEL,),
        )(simp_hbm, q_hbm, out_hbm)

    return sc_kernel(q_w, fx, fy, embs, simp_w)


def kernel(input, anchors, embs, simplices):
    n = input.shape[0]
    p = anchors.shape[0]
    f = embs.shape[1]

    def coord_table(col, cvals):
        a = anchors[:, col]
        pad = (-p) % 128
        t = _tc_tanh(jnp.pad(a, (0, pad)).reshape(-1, 128)).reshape(-1)[:p]
        return jnp.concatenate([t, jnp.asarray(cvals, dtype=input.dtype)])

    fx = coord_table(0, [-1.0, -1.0, 1.0, 1.0])
    fy = coord_table(1, [-1.0, 1.0, -1.0, 1.0])

    simp_w = simplices.reshape(-1, 128)
    q_w = input.reshape(-1, 128)
    out2 = _sc_embed(q_w, fx, fy, embs, simp_w, n, f)
    return out2.reshape(n, f)
